# baseline (speedup 1.0000x reference)
import functools

import jax
import jax.numpy as jnp
from jax import lax
from jax.experimental import pallas as pl
from jax.experimental.pallas import tpu as pltpu

B, N, M = 4, 4096, 4096


def _chamfer_body(p1_ref, x2_ref, y2_ref, out_ref):
    p1 = p1_ref[0, :, :]
    x1 = p1[:, 0:1]
    y1 = p1[:, 1:2]
    x2 = x2_ref[0, 0, :].reshape(1, M)
    y2 = y2_ref[0, 0, :].reshape(1, M)

    ones = jnp.ones((N, 1), jnp.bfloat16)
    am = jnp.concatenate(
        [
            (x1.astype(jnp.bfloat16) * jnp.bfloat16(-2.0)),
            (y1.astype(jnp.bfloat16) * jnp.bfloat16(-2.0)),
            ones,
            ones,
            ones,
        ],
        axis=1,
    )

    b2 = x2 * x2 + y2 * y2
    b2h1 = b2.astype(jnp.bfloat16)
    r1 = b2 - b2h1.astype(jnp.float32)
    b2h2 = r1.astype(jnp.bfloat16)
    b2h3 = (r1 - b2h2.astype(jnp.float32)).astype(jnp.bfloat16)
    bm = jnp.concatenate(
        [x2.astype(jnp.bfloat16), y2.astype(jnp.bfloat16), b2h1, b2h2, b2h3],
        axis=0,
    )

    q = lax.dot_general(
        am, bm, (((1,), (0,)), ((), ())),
        preferred_element_type=jnp.float32,
    )

    a2 = x1 * x1 + y1 * y1
    s = q + a2

    rowmin = jnp.min(s, axis=1)
    colmin = jnp.min(s, axis=0)

    cost = (
        jnp.sum(jnp.maximum(rowmin, 0.0)) * (1.0 / N)
        + jnp.sum(jnp.maximum(colmin, 0.0)) * (1.0 / M)
    )
    out_ref[...] = jnp.full((1, 1, 128), cost, jnp.float32)


@jax.jit
def kernel(points1, points2):
    x2 = points2[..., 0].reshape(B, 1, M)
    y2 = points2[..., 1].reshape(B, 1, M)

    out = pl.pallas_call(
        _chamfer_body,
        grid=(B,),
        in_specs=[
            pl.BlockSpec((1, N, 2), lambda b: (b, 0, 0)),
            pl.BlockSpec((1, 1, M), lambda b: (b, 0, 0)),
            pl.BlockSpec((1, 1, M), lambda b: (b, 0, 0)),
        ],
        out_specs=pl.BlockSpec((1, 1, 128), lambda b: (b, 0, 0)),
        out_shape=jax.ShapeDtypeStruct((B, 1, 128), jnp.float32),
        compiler_params=pltpu.CompilerParams(
            dimension_semantics=("arbitrary",),
        ),
    )(points1, x2, y2)
    return jnp.sum(out[:, 0, 0])


# speedup vs baseline: 1.2436x; 1.2436x over previous
import functools

import jax
import jax.numpy as jnp
from jax import lax
from jax.experimental import pallas as pl
from jax.experimental.pallas import tpu as pltpu

B, N, M = 4, 4096, 4096
BI = 4096
NB = N // BI


def _chamfer_body(x1_ref, y1_ref, x2_ref, y2_ref, out_ref, colmin_ref):
    b = pl.program_id(0)
    ib = pl.program_id(1)

    x1 = x1_ref[0, 0, :].reshape(BI, 1)
    y1 = y1_ref[0, 0, :].reshape(BI, 1)
    x2 = x2_ref[0, 0, :].reshape(1, M)
    y2 = y2_ref[0, 0, :].reshape(1, M)

    ones = jnp.ones((BI, 1), jnp.bfloat16)
    am = jnp.concatenate(
        [
            (x1.astype(jnp.bfloat16) * jnp.bfloat16(-2.0)),
            (y1.astype(jnp.bfloat16) * jnp.bfloat16(-2.0)),
            ones,
            ones,
            ones,
        ],
        axis=1,
    )

    b2 = x2 * x2 + y2 * y2
    b2h1 = b2.astype(jnp.bfloat16)
    r1 = b2 - b2h1.astype(jnp.float32)
    b2h2 = r1.astype(jnp.bfloat16)
    b2h3 = (r1 - b2h2.astype(jnp.float32)).astype(jnp.bfloat16)
    bm = jnp.concatenate(
        [x2.astype(jnp.bfloat16), y2.astype(jnp.bfloat16), b2h1, b2h2, b2h3],
        axis=0,
    )

    q = lax.dot_general(
        am, bm, (((1,), (0,)), ((), ())),
        preferred_element_type=jnp.float32,
    )

    a2 = x1 * x1 + y1 * y1
    s = q + a2

    rowmin = jnp.min(s, axis=1)
    colmin = jnp.min(s, axis=0).reshape(1, M)

    @pl.when(ib == 0)
    def _init_col():
        colmin_ref[...] = colmin

    @pl.when(ib != 0)
    def _acc_col():
        colmin_ref[...] = jnp.minimum(colmin_ref[...], colmin)

    @pl.when(jnp.logical_and(b == 0, ib == 0))
    def _init_out():
        out_ref[0, 0] = 0.0

    partial = jnp.sum(jnp.maximum(rowmin, 0.0)) * (1.0 / N)

    @pl.when(ib == NB - 1)
    def _finish_batch():
        colsum = jnp.sum(jnp.maximum(colmin_ref[...], 0.0))
        out_ref[0, 0] += partial + colsum * (1.0 / M)

    @pl.when(ib != NB - 1)
    def _acc_row():
        out_ref[0, 0] += partial


@jax.jit
def kernel(points1, points2):
    x1 = points1[..., 0].reshape(B * NB, 1, BI)
    y1 = points1[..., 1].reshape(B * NB, 1, BI)
    x2 = points2[..., 0].reshape(B, 1, M)
    y2 = points2[..., 1].reshape(B, 1, M)

    out = pl.pallas_call(
        _chamfer_body,
        grid=(B, NB),
        in_specs=[
            pl.BlockSpec((1, 1, BI), lambda b, i: (b * NB + i, 0, 0)),
            pl.BlockSpec((1, 1, BI), lambda b, i: (b * NB + i, 0, 0)),
            pl.BlockSpec((1, 1, M), lambda b, i: (b, 0, 0)),
            pl.BlockSpec((1, 1, M), lambda b, i: (b, 0, 0)),
        ],
        out_specs=pl.BlockSpec(
            (1, 1), lambda b, i: (0, 0), memory_space=pltpu.SMEM
        ),
        out_shape=jax.ShapeDtypeStruct((1, 1), jnp.float32),
        scratch_shapes=[pltpu.VMEM((1, M), jnp.float32)],
    )(x1, y1, x2, y2)
    return out[0, 0]
